# baseline (speedup 1.0000x reference)
import functools

import jax
import jax.numpy as jnp
from jax import lax
from jax.experimental import pallas as pl
from jax.experimental.pallas import tpu as pltpu
from jax.experimental.pallas import tpu_sc as plsc

N = 10000
NP = 10240
E = 320000
DIN = 128
HID = 256
DOUT = 128

BLK = 1024
NB = NP // BLK

NC = 2
NS = 16
CH = 112
NCHT = 96
NQ = NCHT // 6
EPAD = NC * NS * NCHT * CH
SLAB = NP // NS


def _segsum128(tables, src2d, dst2d, zeros):
    nt = len(tables)
    mesh = plsc.VectorSubcoreMesh(core_axis_name="c", subcore_axis_name="s")

    @functools.partial(
        pl.kernel,
        out_type=jax.ShapeDtypeStruct((nt * 2 * NP, 128), jnp.float32),
        mesh=mesh,
        scratch_types=[
            pltpu.VMEM((NQ, CH), jnp.int32),
            pltpu.VMEM((NQ, CH), jnp.int32),
            pltpu.VMEM((CH, 128), jnp.float32),
            pltpu.VMEM((CH, 128), jnp.float32),
            pltpu.VMEM((CH, 128), jnp.float32),
            pltpu.VMEM_SHARED((NP, 128), jnp.float32),
            pltpu.SemaphoreType.DMA,
            pltpu.SemaphoreType.DMA,
            pltpu.SemaphoreType.DMA,
        ],
    )
    def k(*refs):
        tbls = refs[:nt]
        src_hbm, dst_hbm, zero_hbm, out_hbm = refs[nt:nt + 4]
        srcv, dstv, r0, r1, r2, acc, sem0, sem1, sem2 = refs[nt + 4:]
        rows = (r0, r1, r2)
        sems = (sem0, sem1, sem2)
        c = lax.axis_index("c")
        s = lax.axis_index("s")
        wid = s * NC + c
        for p, tbl in enumerate(tbls):
            pltpu.sync_copy(zero_hbm.at[pl.ds(s * SLAB, SLAB)],
                            acc.at[pl.ds(s * SLAB, SLAB)])
            plsc.subcore_barrier()
            for q in range(6):
                base = wid * NCHT + q * NQ
                pltpu.sync_copy(src_hbm.at[pl.ds(base, NQ)], srcv)
                pltpu.sync_copy(dst_hbm.at[pl.ds(base, NQ)], dstv)
                for b in range(3):
                    pltpu.async_copy(tbl.at[srcv.at[b]], rows[b], sems[b])

                def body(jj, carry, tbl=tbl):
                    j = jj * 3
                    for b in range(3):
                        pltpu.make_async_copy(tbl.at[pl.ds(0, CH)], rows[b],
                                              sems[b]).wait()
                        pltpu.sync_copy(rows[b], acc.at[dstv.at[j + b]],
                                        add=True)

                        @pl.when(j + b + 3 < NQ)
                        def _(b=b):
                            pltpu.async_copy(tbl.at[srcv.at[j + b + 3]],
                                             rows[b], sems[b])

                    return carry

                lax.fori_loop(0, NQ // 3, body, 0)
                for b in range(NQ - 3 * (NQ // 3)):
                    j = 3 * (NQ // 3) + b
                    pltpu.make_async_copy(tbl.at[pl.ds(0, CH)], rows[b],
                                          sems[b]).wait()
                    pltpu.sync_copy(rows[b], acc.at[dstv.at[j]], add=True)
            plsc.subcore_barrier()
            pltpu.sync_copy(acc.at[pl.ds(s * SLAB, SLAB)],
                            out_hbm.at[pl.ds((p * 2 + c) * NP + s * SLAB, SLAB)])
            plsc.subcore_barrier()

    return k(*tables, src2d, dst2d, zeros).reshape(nt * 2, NP, 128)


def _stats_update(st_ref, v, g):
    rows = g * BLK + lax.broadcasted_iota(jnp.int32, (BLK, 1), 0)
    m = (rows < N).astype(jnp.float32)
    vm = v * m
    s0 = jnp.sum(vm, axis=0)[None, :]
    s1 = jnp.sum(vm * v, axis=0)[None, :]
    upd = jnp.concatenate([s0, s1, jnp.zeros((6, v.shape[1]), jnp.float32)], 0)

    @pl.when(g == 0)
    def _():
        st_ref[...] = upd

    @pl.when(g > 0)
    def _():
        st_ref[...] = st_ref[...] + upd


def _mlp(u, w1_ref, b1_ref, w2_ref, b2_ref):
    t = jnp.dot(u, w1_ref[...], preferred_element_type=jnp.float32)
    t = jnp.maximum(t + b1_ref[...], 0.0)
    v = jnp.dot(t, w2_ref[...], preferred_element_type=jnp.float32)
    return v + b2_ref[...]


def _bn_apply(v, st_ref, g_ref, be_ref):
    mu = st_ref[0:1, :] * (1.0 / N)
    var = st_ref[1:2, :] * (1.0 / N) - mu * mu
    inv = lax.rsqrt(var + 1e-5)
    return jnp.maximum((v - mu) * inv * g_ref[...] + be_ref[...], 0.0)


def _fused_layer_body(nin, *refs):
    xs = refs[:nin]
    (p_ref, w1_ref, b1_ref, w2_ref, b2_ref, g_ref, be_ref,
     lo_ref, hi_ref, vs_ref, st_ref) = refs[nin:]
    ph = pl.program_id(0)
    g = pl.program_id(1)

    @pl.when(ph == 0)
    def _():
        if nin == 1:
            u = xs[0][...] + p_ref[0] + p_ref[1]
        else:
            u = jnp.concatenate([xs[0][...] + p_ref[0] + p_ref[1],
                                 xs[1][...] + p_ref[2] + p_ref[3]], axis=1)
        v = _mlp(u, w1_ref, b1_ref, w2_ref, b2_ref)
        vs_ref[pl.ds(g * BLK, BLK), :] = v
        _stats_update(st_ref, v, g)

    @pl.when(ph == 1)
    def _():
        h = _bn_apply(vs_ref[pl.ds(g * BLK, BLK), :], st_ref, g_ref, be_ref)
        lo_ref[...] = h[:, :128]
        hi_ref[...] = h[:, 128:]


def _mlp_final_body(lo_ref, hi_ref, p_ref,
                    w1_ref, b1_ref, w2_ref, b2_ref, o_ref):
    u = jnp.concatenate([lo_ref[...] + p_ref[0] + p_ref[1],
                         hi_ref[...] + p_ref[2] + p_ref[3]], axis=1)
    v = _mlp(u, w1_ref, b1_ref, w2_ref, b2_ref)
    mx = jnp.max(v, axis=1, keepdims=True)
    e = jnp.exp(v - mx)
    lse = jnp.log(jnp.sum(e, axis=1, keepdims=True)) + mx
    o_ref[...] = v - lse


def _row_spec(d):
    return pl.BlockSpec((BLK, d), lambda g: (g, 0))


def _part_spec(n=2):
    return pl.BlockSpec((n, BLK, 128), lambda g: (0, g, 0))


def _full_spec(r, c):
    return pl.BlockSpec((r, c), lambda g: (0, 0))


def _row_spec2(d):
    return pl.BlockSpec((BLK, d), lambda ph, g: (g * (1 - ph), 0))


def _part_spec2(n):
    return pl.BlockSpec((n, BLK, 128), lambda ph, g: (0, g * (1 - ph), 0))


def _out_spec2(d):
    return pl.BlockSpec((BLK, d), lambda ph, g: (g * ph, 0))


def _full_spec2(r, c):
    return pl.BlockSpec((r, c), lambda ph, g: (0, 0))


def _fused_layer(xs, P, W1, b1, W2, b2, gamma, beta):
    nin = len(xs)
    din = DIN if nin == 1 else 128
    return pl.pallas_call(
        functools.partial(_fused_layer_body, nin),
        grid=(2, NB),
        in_specs=[_row_spec2(din)] * nin + [
            _part_spec2(2 * nin), _full_spec2(nin * 128, HID),
            _full_spec2(1, HID), _full_spec2(HID, HID), _full_spec2(1, HID),
            _full_spec2(1, HID), _full_spec2(1, HID)],
        out_specs=[_out_spec2(128), _out_spec2(128)],
        out_shape=[jax.ShapeDtypeStruct((NP, 128), jnp.float32),
                   jax.ShapeDtypeStruct((NP, 128), jnp.float32)],
        scratch_shapes=[pltpu.VMEM((NP, HID), jnp.float32),
                        pltpu.VMEM((8, HID), jnp.float32)],
    )(*xs, P, W1, b1, W2, b2, gamma, beta)


def _mlp_final(lo, hi, P, W1, b1, W2, b2):
    return pl.pallas_call(
        _mlp_final_body,
        grid=(NB,),
        in_specs=[_row_spec(128), _row_spec(128), _part_spec(4),
                  _full_spec(HID, HID), _full_spec(1, HID),
                  _full_spec(HID, DOUT), _full_spec(1, DOUT)],
        out_specs=_row_spec(DOUT),
        out_shape=jax.ShapeDtypeStruct((N, DOUT), jnp.float32),
    )(lo, hi, P, W1, b1, W2, b2)


def kernel(x, edge_index, W1_0, b1_0, W2_0, b2_0, W1_1, b1_1, W2_1, b2_1,
           W1_2, b1_2, W2_2, b2_2, g_0, be_0, g_1, be_1):
    src = edge_index[0]
    dst = edge_index[1]
    zeros = jnp.zeros((NP, 128), jnp.float32)
    xp = jnp.zeros((NP, DIN), jnp.float32).at[:N].set(x)
    ar = jnp.arange(EPAD - E, dtype=jnp.int32)
    src2d = jnp.concatenate([src, ar % N]).reshape(-1, CH)
    dst2d = jnp.concatenate([dst, N + ar % (NP - N)]).reshape(-1, CH)

    r = lambda b: b.reshape(1, -1)

    P = _segsum128((xp,), src2d, dst2d, zeros)
    lo, hi = _fused_layer((xp,), P, W1_0, r(b1_0), W2_0, r(b2_0),
                          r(g_0), r(be_0))

    P = _segsum128((lo, hi), src2d, dst2d, zeros)
    lo, hi = _fused_layer((lo, hi), P, W1_1, r(b1_1), W2_1, r(b2_1),
                          r(g_1), r(be_1))

    P = _segsum128((lo, hi), src2d, dst2d, zeros)
    return _mlp_final(lo, hi, P, W1_2, r(b1_2), W2_2, r(b2_2))


# speedup vs baseline: 1.0554x; 1.0554x over previous
import functools

import jax
import jax.numpy as jnp
from jax import lax
from jax.experimental import pallas as pl
from jax.experimental.pallas import tpu as pltpu
from jax.experimental.pallas import tpu_sc as plsc

N = 10000
NP = 10240
E = 320000
DIN = 128
HID = 256
DOUT = 128

BLK = 1024
NB = NP // BLK

NC = 2
NS = 16
CH = 128
NCHT = 80
NQ = NCHT // 2
EPAD = NC * NS * NCHT * CH
SLAB = NP // NS


def _segsum128(tables, src2d, dst2d, zeros):
    nt = len(tables)
    mesh = plsc.VectorSubcoreMesh(core_axis_name="c", subcore_axis_name="s")

    @functools.partial(
        pl.kernel,
        out_type=jax.ShapeDtypeStruct((nt * 2 * NP, 128), jnp.float32),
        mesh=mesh,
        scratch_types=[
            pltpu.VMEM((NQ, CH), jnp.int32),
            pltpu.VMEM((NQ, CH), jnp.int32),
            pltpu.VMEM((CH, 128), jnp.float32),
            pltpu.VMEM((CH, 128), jnp.float32),
            pltpu.VMEM_SHARED((NP, 128), jnp.float32),
            pltpu.SemaphoreType.DMA,
            pltpu.SemaphoreType.DMA,
        ],
    )
    def k(*refs):
        tbls = refs[:nt]
        src_hbm, dst_hbm, zero_hbm, out_hbm = refs[nt:nt + 4]
        srcv, dstv, rows0, rows1, acc, sem0, sem1 = refs[nt + 4:]
        c = lax.axis_index("c")
        s = lax.axis_index("s")
        wid = s * NC + c
        for p, tbl in enumerate(tbls):
            pltpu.sync_copy(zero_hbm.at[pl.ds(s * SLAB, SLAB)],
                            acc.at[pl.ds(s * SLAB, SLAB)])
            plsc.subcore_barrier()
            for q in range(2):
                base = wid * NCHT + q * NQ
                pltpu.sync_copy(src_hbm.at[pl.ds(base, NQ)], srcv)
                pltpu.sync_copy(dst_hbm.at[pl.ds(base, NQ)], dstv)
                pltpu.async_copy(tbl.at[srcv.at[0]], rows0, sem0)
                pltpu.async_copy(tbl.at[srcv.at[1]], rows1, sem1)

                def body(jj, carry, tbl=tbl):
                    j = jj * 2
                    pltpu.make_async_copy(tbl.at[pl.ds(0, CH)], rows0,
                                          sem0).wait()
                    pltpu.sync_copy(rows0, acc.at[dstv.at[j]], add=True)

                    @pl.when(j + 2 < NQ)
                    def _():
                        pltpu.async_copy(tbl.at[srcv.at[j + 2]], rows0, sem0)

                    pltpu.make_async_copy(tbl.at[pl.ds(0, CH)], rows1,
                                          sem1).wait()
                    pltpu.sync_copy(rows1, acc.at[dstv.at[j + 1]], add=True)

                    @pl.when(j + 3 < NQ)
                    def _():
                        pltpu.async_copy(tbl.at[srcv.at[j + 3]], rows1, sem1)

                    return carry

                lax.fori_loop(0, NQ // 2, body, 0)
            plsc.subcore_barrier()
            pltpu.sync_copy(acc.at[pl.ds(s * SLAB, SLAB)],
                            out_hbm.at[pl.ds((p * 2 + c) * NP + s * SLAB, SLAB)])
            plsc.subcore_barrier()

    return k(*tables, src2d, dst2d, zeros).reshape(nt * 2, NP, 128)


def _stats_update(st_ref, v, g):
    rows = g * BLK + lax.broadcasted_iota(jnp.int32, (BLK, 1), 0)
    m = (rows < N).astype(jnp.float32)
    vm = v * m
    s0 = jnp.sum(vm, axis=0)[None, :]
    s1 = jnp.sum(vm * v, axis=0)[None, :]
    upd = jnp.concatenate([s0, s1, jnp.zeros((6, v.shape[1]), jnp.float32)], 0)

    @pl.when(g == 0)
    def _():
        st_ref[...] = upd

    @pl.when(g > 0)
    def _():
        st_ref[...] = st_ref[...] + upd


def _mlp(u, w1_ref, b1_ref, w2_ref, b2_ref):
    t = jnp.dot(u, w1_ref[...], preferred_element_type=jnp.float32)
    t = jnp.maximum(t + b1_ref[...], 0.0)
    v = jnp.dot(t, w2_ref[...], preferred_element_type=jnp.float32)
    return v + b2_ref[...]


def _bn_apply(v, st_ref, g_ref, be_ref):
    mu = st_ref[0:1, :] * (1.0 / N)
    var = st_ref[1:2, :] * (1.0 / N) - mu * mu
    inv = lax.rsqrt(var + 1e-5)
    return jnp.maximum((v - mu) * inv * g_ref[...] + be_ref[...], 0.0)


def _fused_layer_body(nin, *refs):
    xs = refs[:nin]
    (p_ref, w1_ref, b1_ref, w2_ref, b2_ref, g_ref, be_ref,
     lo_ref, hi_ref, vs_ref, st_ref) = refs[nin:]
    ph = pl.program_id(0)
    g = pl.program_id(1)

    @pl.when(ph == 0)
    def _():
        if nin == 1:
            u = xs[0][...] + p_ref[0] + p_ref[1]
        else:
            u = jnp.concatenate([xs[0][...] + p_ref[0] + p_ref[1],
                                 xs[1][...] + p_ref[2] + p_ref[3]], axis=1)
        v = _mlp(u, w1_ref, b1_ref, w2_ref, b2_ref)
        vs_ref[pl.ds(g * BLK, BLK), :] = v
        _stats_update(st_ref, v, g)

    @pl.when(ph == 1)
    def _():
        h = _bn_apply(vs_ref[pl.ds(g * BLK, BLK), :], st_ref, g_ref, be_ref)
        lo_ref[...] = h[:, :128]
        hi_ref[...] = h[:, 128:]


def _mlp_final_body(lo_ref, hi_ref, p_ref,
                    w1_ref, b1_ref, w2_ref, b2_ref, o_ref):
    u = jnp.concatenate([lo_ref[...] + p_ref[0] + p_ref[1],
                         hi_ref[...] + p_ref[2] + p_ref[3]], axis=1)
    v = _mlp(u, w1_ref, b1_ref, w2_ref, b2_ref)
    mx = jnp.max(v, axis=1, keepdims=True)
    e = jnp.exp(v - mx)
    lse = jnp.log(jnp.sum(e, axis=1, keepdims=True)) + mx
    o_ref[...] = v - lse


def _row_spec(d):
    return pl.BlockSpec((BLK, d), lambda g: (g, 0))


def _part_spec(n=2):
    return pl.BlockSpec((n, BLK, 128), lambda g: (0, g, 0))


def _full_spec(r, c):
    return pl.BlockSpec((r, c), lambda g: (0, 0))


def _row_spec2(d):
    return pl.BlockSpec((BLK, d), lambda ph, g: (g * (1 - ph), 0))


def _part_spec2(n):
    return pl.BlockSpec((n, BLK, 128), lambda ph, g: (0, g * (1 - ph), 0))


def _out_spec2(d):
    return pl.BlockSpec((BLK, d), lambda ph, g: (g * ph, 0))


def _full_spec2(r, c):
    return pl.BlockSpec((r, c), lambda ph, g: (0, 0))


def _fused_layer(xs, P, W1, b1, W2, b2, gamma, beta):
    nin = len(xs)
    din = DIN if nin == 1 else 128
    return pl.pallas_call(
        functools.partial(_fused_layer_body, nin),
        grid=(2, NB),
        in_specs=[_row_spec2(din)] * nin + [
            _part_spec2(2 * nin), _full_spec2(nin * 128, HID),
            _full_spec2(1, HID), _full_spec2(HID, HID), _full_spec2(1, HID),
            _full_spec2(1, HID), _full_spec2(1, HID)],
        out_specs=[_out_spec2(128), _out_spec2(128)],
        out_shape=[jax.ShapeDtypeStruct((NP, 128), jnp.float32),
                   jax.ShapeDtypeStruct((NP, 128), jnp.float32)],
        scratch_shapes=[pltpu.VMEM((NP, HID), jnp.float32),
                        pltpu.VMEM((8, HID), jnp.float32)],
    )(*xs, P, W1, b1, W2, b2, gamma, beta)


def _mlp_final(lo, hi, P, W1, b1, W2, b2):
    return pl.pallas_call(
        _mlp_final_body,
        grid=(NB,),
        in_specs=[_row_spec(128), _row_spec(128), _part_spec(4),
                  _full_spec(HID, HID), _full_spec(1, HID),
                  _full_spec(HID, DOUT), _full_spec(1, DOUT)],
        out_specs=_row_spec(DOUT),
        out_shape=jax.ShapeDtypeStruct((N, DOUT), jnp.float32),
    )(lo, hi, P, W1, b1, W2, b2)


def kernel(x, edge_index, W1_0, b1_0, W2_0, b2_0, W1_1, b1_1, W2_1, b2_1,
           W1_2, b1_2, W2_2, b2_2, g_0, be_0, g_1, be_1):
    src = edge_index[0]
    dst = edge_index[1]
    zeros = jnp.zeros((NP, 128), jnp.float32)
    xp = jnp.zeros((NP, DIN), jnp.float32).at[:N].set(x)
    ar = jnp.arange(EPAD - E, dtype=jnp.int32)
    src2d = jnp.concatenate([src, ar % N]).reshape(-1, CH)
    dst2d = jnp.concatenate([dst, N + ar % (NP - N)]).reshape(-1, CH)

    r = lambda b: b.reshape(1, -1)

    P = _segsum128((xp,), src2d, dst2d, zeros)
    lo, hi = _fused_layer((xp,), P, W1_0, r(b1_0), W2_0, r(b2_0),
                          r(g_0), r(be_0))

    P = _segsum128((lo, hi), src2d, dst2d, zeros)
    lo, hi = _fused_layer((lo, hi), P, W1_1, r(b1_1), W2_1, r(b2_1),
                          r(g_1), r(be_1))

    P = _segsum128((lo, hi), src2d, dst2d, zeros)
    return _mlp_final(lo, hi, P, W1_2, r(b1_2), W2_2, r(b2_2))
